# baseline (speedup 1.0000x reference)
import functools

import jax
import jax.numpy as jnp
from jax import lax
from jax.experimental import pallas as pl
from jax.experimental.pallas import tpu as pltpu
from jax.experimental.pallas import tpu_sc as plsc

N_NODES = 50000
N_EDGES = 800000
NP = 51200
EP = 851968
CH = 1024
NSP = 50016
RPT = NSP // 16
HRPT = RPT // 2
F32 = jnp.float32


def _f(shape):
    return jax.ShapeDtypeStruct(shape, F32)


@functools.lru_cache(maxsize=None)
def _build():
    info = plsc.get_sparse_core_info()
    NC, NS = info.num_cores, info.num_subcores
    NW = NC * NS
    mesh = plsc.VectorSubcoreMesh(core_axis_name="c", subcore_axis_name="s")
    CP = pltpu.CompilerParams(use_tc_tiling_on_sc=False)

    def k1_body(x_ref, w_ref, aps_ref, apd_ref,
                h0_ref, h1_ref, h2_ref, h3_ref, as_ref, ad_ref):
        h = jnp.dot(x_ref[...], w_ref[...], preferred_element_type=F32)
        h0_ref[...] = h[:, 0:16]
        h1_ref[...] = h[:, 16:32]
        h2_ref[...] = h[:, 32:48]
        h3_ref[...] = h[:, 48:64]
        as_ref[...] = jnp.dot(h, aps_ref[...], preferred_element_type=F32)
        ad_ref[...] = jnp.dot(h, apd_ref[...], preferred_element_type=F32)

    blk16 = pl.BlockSpec((512, 16), lambda i: (i, 0))
    k1 = pl.pallas_call(
        k1_body,
        grid=(NP // 512,),
        in_specs=[
            pl.BlockSpec((512, 300), lambda i: (i, 0)),
            pl.BlockSpec((300, 64), lambda i: (0, 0)),
            pl.BlockSpec((64, 16), lambda i: (0, 0)),
            pl.BlockSpec((64, 16), lambda i: (0, 0)),
        ],
        out_specs=[blk16] * 6,
        out_shape=[_f((NP, 16))] * 6,
    )

    def k2_body(o0_ref, o1_ref, o2_ref, o3_ref, b_ref, w_ref, aps_ref,
                apd_ref, h0_ref, h1_ref, h2_ref, h3_ref, as_ref, ad_ref):
        h = jnp.concatenate(
            [o0_ref[...], o1_ref[...], o2_ref[...], o3_ref[...]], axis=1)
        h = h + b_ref[...]
        z = jnp.where(h > 0, h, jnp.exp(h) - 1.0)
        h2 = jnp.dot(z, w_ref[...], preferred_element_type=F32)
        h0_ref[...] = h2[:, 0:16]
        h1_ref[...] = h2[:, 16:32]
        h2_ref[...] = h2[:, 32:48]
        h3_ref[...] = h2[:, 48:64]
        as_ref[...] = jnp.dot(h2, aps_ref[...], preferred_element_type=F32)
        ad_ref[...] = jnp.dot(h2, apd_ref[...], preferred_element_type=F32)

    k2 = pl.pallas_call(
        k2_body,
        grid=(NP // 512,),
        in_specs=[
            blk16, blk16, blk16, blk16,
            pl.BlockSpec((1, 64), lambda i: (0, 0)),
            pl.BlockSpec((64, 64), lambda i: (0, 0)),
            pl.BlockSpec((64, 16), lambda i: (0, 0)),
            pl.BlockSpec((64, 16), lambda i: (0, 0)),
        ],
        out_specs=[blk16] * 6,
        out_shape=[_f((NP, 16))] * 6,
    )

    def kc_body(d0_ref, d1_ref, rd_ref):
        rd_ref[...] = 1.0 / (d0_ref[...] + d1_ref[...] + 1e-16)

    kcomb = pl.pallas_call(
        kc_body,
        grid=(NP // 2048,),
        in_specs=[pl.BlockSpec((2048, 16), lambda i: (i, 0))] * 2,
        out_specs=pl.BlockSpec((2048, 16), lambda i: (i, 0)),
        out_shape=_f((NP, 16)),
    )

    EPW = EP // NW
    NCHD = EPW // CH

    @functools.partial(
        pl.kernel, mesh=mesh, compiler_params=CP,
        out_type=(_f((NP, 16)), _f((NP, 16))),
        scratch_types=[
            pltpu.VMEM((CH,), jnp.int32),
            pltpu.VMEM((CH,), jnp.int32),
            pltpu.VMEM((CH, 16), F32),
            pltpu.VMEM((CH, 16), F32),
            pltpu.VMEM((CH, 16), F32),
            pltpu.VMEM((HRPT, 16), F32),
            pltpu.VMEM_SHARED((NSP, 16), F32),
            pltpu.SemaphoreType.DMA,
        ],
    )
    def sc_den(src_hbm, dst_hbm, as_hbm, ad_hbm, d0_hbm, d1_hbm,
               sidx, didx, rs, rd, wv, stage, shared, sem):
        cid = lax.axis_index("c")
        sid = lax.axis_index("s")
        wid = sid * NC + cid
        zero16 = jnp.zeros((16,), F32)

        def zrow(i, _):
            stage[i, :] = zero16
            return 0
        lax.fori_loop(0, HRPT, zrow, 0)
        pltpu.sync_copy(stage, shared.at[pl.ds(sid * RPT, HRPT)])
        pltpu.sync_copy(stage, shared.at[pl.ds(sid * RPT + HRPT, HRPT)])
        plsc.subcore_barrier()

        def chunk(g, _):
            base = wid * EPW + g * CH
            pltpu.sync_copy(src_hbm.at[pl.ds(base, CH)], sidx)
            pltpu.sync_copy(dst_hbm.at[pl.ds(base, CH)], didx)
            c1 = pltpu.async_copy(as_hbm.at[sidx], rs, sem)
            c2 = pltpu.async_copy(ad_hbm.at[didx], rd, sem)
            c1.wait()
            c2.wait()

            def edge(i, _):
                e = rs[i, :] + rd[i, :]
                e = jnp.where(e < 0, e * jnp.float32(0.2), e)
                wv[i, :] = jnp.exp(e)
                return 0
            lax.fori_loop(0, CH, edge, 0, unroll=8)
            pltpu.sync_copy(wv, shared.at[didx], add=True)
            return 0
        lax.fori_loop(0, NCHD, chunk, 0)
        plsc.subcore_barrier()

        def dump(half, _):
            r0 = sid * RPT + half * HRPT
            pltpu.sync_copy(shared.at[pl.ds(r0, HRPT)], stage)

            @pl.when(cid == 0)
            def _():
                pltpu.sync_copy(stage, d0_hbm.at[pl.ds(r0, HRPT)])

            @pl.when(cid == 1)
            def _():
                pltpu.sync_copy(stage, d1_hbm.at[pl.ds(r0, HRPT)])
            return 0
        lax.fori_loop(0, 2, dump, 0)

    @functools.partial(
        pl.kernel, mesh=mesh, compiler_params=CP,
        out_type=_f((EP, 16)),
        scratch_types=[
            pltpu.VMEM((CH,), jnp.int32),
            pltpu.VMEM((CH,), jnp.int32),
            pltpu.VMEM((CH, 16), F32),
            pltpu.VMEM((CH, 16), F32),
            pltpu.VMEM((CH, 16), F32),
            pltpu.VMEM((CH, 16), F32),
            pltpu.VMEM((CH, 16), F32),
            pltpu.SemaphoreType.DMA,
        ],
    )
    def sc_alpha(src_hbm, dst_hbm, as_hbm, ad_hbm, d0_hbm, d1_hbm, al_hbm,
                 sidx, didx, rs, rd, rdn, dn1, alv, sem):
        cid = lax.axis_index("c")
        sid = lax.axis_index("s")
        wid = sid * NC + cid

        def chunk(g, _):
            base = wid * EPW + g * CH
            pltpu.sync_copy(src_hbm.at[pl.ds(base, CH)], sidx)
            pltpu.sync_copy(dst_hbm.at[pl.ds(base, CH)], didx)
            c1 = pltpu.async_copy(as_hbm.at[sidx], rs, sem)
            c2 = pltpu.async_copy(ad_hbm.at[didx], rd, sem)
            c3 = pltpu.async_copy(d0_hbm.at[didx], rdn, sem)
            c4 = pltpu.async_copy(d1_hbm.at[didx], dn1, sem)
            c1.wait()
            c2.wait()
            c3.wait()
            c4.wait()

            def edge(i, _):
                e = rs[i, :] + rd[i, :]
                e = jnp.where(e < 0, e * jnp.float32(0.2), e)
                den = rdn[i, :] + dn1[i, :] + jnp.float32(1e-16)
                alv[i, :] = jnp.exp(e) / den
                return 0
            lax.fori_loop(0, CH, edge, 0, unroll=8)
            pltpu.sync_copy(alv, al_hbm.at[pl.ds(base, CH)])
            return 0
        lax.fori_loop(0, NCHD, chunk, 0)

    EPT = EP // NS
    NCHM = EPT // CH

    def sc_msg_body(H, src_hbm, dst_hbm, al_hbm, h0_hbm, h1_hbm, h2_hbm,
                    h3_hbm, bias_hbm, o0_hbm, o1_hbm, o2_hbm, o3_hbm,
                    sidx, didx, alv, hv, msg, stage, bvec,
                    shared, sem, sem2):
        cid = lax.axis_index("c")
        sid = lax.axis_index("s")
        zero16 = jnp.zeros((16,), F32)
        msk8 = lax.iota(jnp.int32, 16) < 8
        pltpu.sync_copy(bias_hbm, bvec)

        def qpass(q, h_hbm, o_hbm):
            def zrow(i, _):
                stage[i, :] = zero16
                return 0
            lax.fori_loop(0, HRPT, zrow, 0)
            pltpu.sync_copy(stage, shared.at[pl.ds(sid * RPT, HRPT)])
            pltpu.sync_copy(stage, shared.at[pl.ds(sid * RPT + HRPT, HRPT)])
            plsc.subcore_barrier()

            def chunk(g, _):
                base = sid * EPT + g * CH
                pltpu.sync_copy(src_hbm.at[pl.ds(base, CH)], sidx)
                pltpu.sync_copy(dst_hbm.at[pl.ds(base, CH)], didx)
                c1 = pltpu.async_copy(al_hbm.at[pl.ds(base, CH)], alv, sem)
                c4 = pltpu.async_copy(h_hbm.at[sidx], hv, sem2)
                c1.wait()
                c4.wait()

                def edge(i, _):
                    al = alv[i, :]
                    if H == 8:
                        a_lo = al[2 * q]
                        a_hi = al[2 * q + 1]
                        m = jnp.where(msk8, jnp.full((16,), a_lo, F32),
                                      jnp.full((16,), a_hi, F32))
                    else:
                        m = jnp.full((16,), al[0], F32)
                    msg[i, :] = hv[i, :] * m
                    return 0
                lax.fori_loop(0, CH, edge, 0, unroll=8)
                pltpu.sync_copy(msg, shared.at[didx], add=True)
                return 0
            lax.fori_loop(0, NCHM, chunk, 0)
            plsc.subcore_barrier()

            bq = bvec[pl.ds(16 * q, 16)]

            def dump(half, _):
                r0 = sid * RPT + half * HRPT
                pltpu.sync_copy(shared.at[pl.ds(r0, HRPT)], stage)

                def brow(i, _):
                    stage[i, :] = stage[i, :] + bq
                    return 0
                lax.fori_loop(0, HRPT, brow, 0)
                pltpu.sync_copy(stage, o_hbm.at[pl.ds(r0, HRPT)])
                return 0
            lax.fori_loop(0, 2, dump, 0)

        @pl.when(cid == 0)
        def _():
            qpass(0, h0_hbm, o0_hbm)
            qpass(1, h1_hbm, o1_hbm)

        @pl.when(cid == 1)
        def _():
            qpass(2, h2_hbm, o2_hbm)
            qpass(3, h3_hbm, o3_hbm)

    def make_msg(H):
        return functools.partial(
            pl.kernel, mesh=mesh, compiler_params=CP,
            out_type=(_f((NP, 16)),) * 4,
            scratch_types=[
                pltpu.VMEM((CH,), jnp.int32),
                pltpu.VMEM((CH,), jnp.int32),
                pltpu.VMEM((CH, 16), F32),
                pltpu.VMEM((CH, 16), F32),
                pltpu.VMEM((CH, 16), F32),
                pltpu.VMEM((HRPT, 16), F32),
                pltpu.VMEM((64,), F32),
                pltpu.VMEM_SHARED((NSP, 16), F32),
                pltpu.SemaphoreType.DMA,
                pltpu.SemaphoreType.DMA,
            ],
        )(functools.partial(sc_msg_body, H))

    sc_msg8 = make_msg(8)
    sc_msg1 = make_msg(1)

    return k1, k2, sc_den, sc_alpha, sc_msg8, sc_msg1


def _expand_heads(a):
    h = a.shape[0]
    if h == 1:
        m = jnp.tile(a.reshape(-1, 1), (1, 8))
    else:
        eye = jnp.eye(h, dtype=F32)
        m = (eye[:, None, :] * a[:, :, None]).reshape(-1, h)
    return jnp.concatenate([m, jnp.zeros_like(m)], axis=1)


def kernel(x, edge_index, W1, a_src1, a_dst1, b1, W2, a_src2, a_dst2, b2):
    k1, k2, sc_den, sc_alpha, sc_msg8, sc_msg1 = _build()
    N, E = N_NODES, N_EDGES

    src = edge_index[0].astype(jnp.int32)
    dst = edge_index[1].astype(jnp.int32)
    loop = jnp.arange(N, dtype=jnp.int32)
    npad = EP - E - N
    srcp = jnp.concatenate([src, loop, jnp.zeros((npad,), jnp.int32)])
    dstp = jnp.concatenate([dst, loop, jnp.full((npad,), N, jnp.int32)])
    xp = jnp.pad(x, ((0, NP - N), (0, 0)))

    h0, h1, h2, h3, as1, ad1 = k1(xp, W1, _expand_heads(a_src1),
                                  _expand_heads(a_dst1))
    d0, d1 = sc_den(srcp, dstp, as1, ad1)
    al1 = sc_alpha(srcp, dstp, as1, ad1, d0, d1)
    zeros64 = jnp.zeros((64,), F32)
    o0, o1, o2, o3 = sc_msg8(srcp, dstp, al1, h0, h1, h2, h3, zeros64)

    g0, g1, g2, g3, as2, ad2 = k2(o0, o1, o2, o3, b1.reshape(1, 64), W2,
                                  _expand_heads(a_src2), _expand_heads(a_dst2))
    d0b, d1b = sc_den(srcp, dstp, as2, ad2)
    al2 = sc_alpha(srcp, dstp, as2, ad2, d0b, d1b)
    p0, p1, p2, p3 = sc_msg1(srcp, dstp, al2, g0, g1, g2, g3, b2)

    return jnp.concatenate([p0[:N], p1[:N], p2[:N], p3[:N]], axis=1)


# speedup vs baseline: 1.3440x; 1.3440x over previous
import functools

import jax
import jax.numpy as jnp
from jax import lax
from jax.experimental import pallas as pl
from jax.experimental.pallas import tpu as pltpu
from jax.experimental.pallas import tpu_sc as plsc

N_NODES = 50000
N_EDGES = 800000
NP = 51200
EP = 851968
CH = 1024
NSP = 50016
RPT = NSP // 16
HRPT = RPT // 2
F32 = jnp.float32


def _f(shape):
    return jax.ShapeDtypeStruct(shape, F32)


@functools.lru_cache(maxsize=None)
def _build():
    info = plsc.get_sparse_core_info()
    NC, NS = info.num_cores, info.num_subcores
    NW = NC * NS
    mesh = plsc.VectorSubcoreMesh(core_axis_name="c", subcore_axis_name="s")
    CP = pltpu.CompilerParams(use_tc_tiling_on_sc=False)

    def k1_body(x_ref, w_ref, aps_ref, apd_ref,
                h0_ref, h1_ref, h2_ref, h3_ref, as_ref, ad_ref):
        h = jnp.dot(x_ref[...], w_ref[...], preferred_element_type=F32)
        h0_ref[...] = h[:, 0:16]
        h1_ref[...] = h[:, 16:32]
        h2_ref[...] = h[:, 32:48]
        h3_ref[...] = h[:, 48:64]
        as_ref[...] = jnp.dot(h, aps_ref[...], preferred_element_type=F32)
        ad_ref[...] = jnp.dot(h, apd_ref[...], preferred_element_type=F32)

    blk16 = pl.BlockSpec((512, 16), lambda i: (i, 0))
    k1 = pl.pallas_call(
        k1_body,
        grid=(NP // 512,),
        in_specs=[
            pl.BlockSpec((512, 300), lambda i: (i, 0)),
            pl.BlockSpec((300, 64), lambda i: (0, 0)),
            pl.BlockSpec((64, 16), lambda i: (0, 0)),
            pl.BlockSpec((64, 16), lambda i: (0, 0)),
        ],
        out_specs=[blk16] * 6,
        out_shape=[_f((NP, 16))] * 6,
    )

    def k2_body(o0_ref, o1_ref, o2_ref, o3_ref, b_ref, w_ref, aps_ref,
                apd_ref, h0_ref, h1_ref, h2_ref, h3_ref, as_ref, ad_ref):
        h = jnp.concatenate(
            [o0_ref[...], o1_ref[...], o2_ref[...], o3_ref[...]], axis=1)
        h = h + b_ref[...]
        z = jnp.where(h > 0, h, jnp.exp(h) - 1.0)
        h2 = jnp.dot(z, w_ref[...], preferred_element_type=F32)
        h0_ref[...] = h2[:, 0:16]
        h1_ref[...] = h2[:, 16:32]
        h2_ref[...] = h2[:, 32:48]
        h3_ref[...] = h2[:, 48:64]
        as_ref[...] = jnp.dot(h2, aps_ref[...], preferred_element_type=F32)
        ad_ref[...] = jnp.dot(h2, apd_ref[...], preferred_element_type=F32)

    k2 = pl.pallas_call(
        k2_body,
        grid=(NP // 512,),
        in_specs=[
            blk16, blk16, blk16, blk16,
            pl.BlockSpec((1, 64), lambda i: (0, 0)),
            pl.BlockSpec((64, 64), lambda i: (0, 0)),
            pl.BlockSpec((64, 16), lambda i: (0, 0)),
            pl.BlockSpec((64, 16), lambda i: (0, 0)),
        ],
        out_specs=[blk16] * 6,
        out_shape=[_f((NP, 16))] * 6,
    )

    def kc_body(d0_ref, d1_ref, rd_ref):
        rd_ref[...] = 1.0 / (d0_ref[...] + d1_ref[...] + 1e-16)

    kcomb = pl.pallas_call(
        kc_body,
        grid=(NP // 2048,),
        in_specs=[pl.BlockSpec((2048, 16), lambda i: (i, 0))] * 2,
        out_specs=pl.BlockSpec((2048, 16), lambda i: (i, 0)),
        out_shape=_f((NP, 16)),
    )

    EPW = EP // NW
    NCHD = EPW // CH

    @functools.partial(
        pl.kernel, mesh=mesh, compiler_params=CP,
        out_type=(_f((NP, 16)), _f((NP, 16))),
        scratch_types=[
            pltpu.VMEM((CH,), jnp.int32),
            pltpu.VMEM((CH,), jnp.int32),
            pltpu.VMEM((CH, 16), F32),
            pltpu.VMEM((CH, 16), F32),
            pltpu.VMEM((CH, 16), F32),
            pltpu.VMEM((HRPT, 16), F32),
            pltpu.VMEM_SHARED((NSP, 16), F32),
            pltpu.SemaphoreType.DMA,
        ],
    )
    def sc_den(src_hbm, dst_hbm, as_hbm, ad_hbm, d0_hbm, d1_hbm,
               sidx, didx, rs, rd, wv, stage, shared, sem):
        cid = lax.axis_index("c")
        sid = lax.axis_index("s")
        wid = sid * NC + cid
        zero16 = jnp.zeros((16,), F32)

        def zrow(i, _):
            stage[i, :] = zero16
            return 0
        lax.fori_loop(0, HRPT, zrow, 0)
        pltpu.sync_copy(stage, shared.at[pl.ds(sid * RPT, HRPT)])
        pltpu.sync_copy(stage, shared.at[pl.ds(sid * RPT + HRPT, HRPT)])
        plsc.subcore_barrier()

        def chunk(g, _):
            base = wid * EPW + g * CH
            pltpu.sync_copy(src_hbm.at[pl.ds(base, CH)], sidx)
            pltpu.sync_copy(dst_hbm.at[pl.ds(base, CH)], didx)
            c1 = pltpu.async_copy(as_hbm.at[sidx], rs, sem)
            c2 = pltpu.async_copy(ad_hbm.at[didx], rd, sem)
            c1.wait()
            c2.wait()

            def edge(i, _):
                e = rs[i, :] + rd[i, :]
                e = jnp.where(e < 0, e * jnp.float32(0.2), e)
                wv[i, :] = jnp.exp(e)
                return 0
            lax.fori_loop(0, CH, edge, 0)
            pltpu.sync_copy(wv, shared.at[didx], add=True)
            return 0
        lax.fori_loop(0, NCHD, chunk, 0)
        plsc.subcore_barrier()

        def dump(half, _):
            r0 = sid * RPT + half * HRPT
            pltpu.sync_copy(shared.at[pl.ds(r0, HRPT)], stage)

            @pl.when(cid == 0)
            def _():
                pltpu.sync_copy(stage, d0_hbm.at[pl.ds(r0, HRPT)])

            @pl.when(cid == 1)
            def _():
                pltpu.sync_copy(stage, d1_hbm.at[pl.ds(r0, HRPT)])
            return 0
        lax.fori_loop(0, 2, dump, 0)

    @functools.partial(
        pl.kernel, mesh=mesh, compiler_params=CP,
        out_type=_f((EP, 16)),
        scratch_types=[
            pltpu.VMEM((CH,), jnp.int32),
            pltpu.VMEM((CH,), jnp.int32),
            pltpu.VMEM((CH, 16), F32),
            pltpu.VMEM((CH, 16), F32),
            pltpu.VMEM((CH, 16), F32),
            pltpu.VMEM((CH, 16), F32),
            pltpu.VMEM((CH, 16), F32),
            pltpu.SemaphoreType.DMA,
        ],
    )
    def sc_alpha(src_hbm, dst_hbm, as_hbm, ad_hbm, d0_hbm, d1_hbm, al_hbm,
                 sidx, didx, rs, rd, rdn, dn1, alv, sem):
        cid = lax.axis_index("c")
        sid = lax.axis_index("s")
        wid = sid * NC + cid

        def chunk(g, _):
            base = wid * EPW + g * CH
            pltpu.sync_copy(src_hbm.at[pl.ds(base, CH)], sidx)
            pltpu.sync_copy(dst_hbm.at[pl.ds(base, CH)], didx)
            c1 = pltpu.async_copy(as_hbm.at[sidx], rs, sem)
            c2 = pltpu.async_copy(ad_hbm.at[didx], rd, sem)
            c3 = pltpu.async_copy(d0_hbm.at[didx], rdn, sem)
            c4 = pltpu.async_copy(d1_hbm.at[didx], dn1, sem)
            c1.wait()
            c2.wait()
            c3.wait()
            c4.wait()

            def edge(i, _):
                e = rs[i, :] + rd[i, :]
                e = jnp.where(e < 0, e * jnp.float32(0.2), e)
                den = rdn[i, :] + dn1[i, :] + jnp.float32(1e-16)
                alv[i, :] = jnp.exp(e) / den
                return 0
            lax.fori_loop(0, CH, edge, 0)
            pltpu.sync_copy(alv, al_hbm.at[pl.ds(base, CH)])
            return 0
        lax.fori_loop(0, NCHD, chunk, 0)

    EPT = EP // NS
    NCHM = EPT // CH

    def sc_msg_body(H, src_hbm, dst_hbm, al_hbm, h0_hbm, h1_hbm, h2_hbm,
                    h3_hbm, bias_hbm, o0_hbm, o1_hbm, o2_hbm, o3_hbm,
                    sidx, didx, alv, hv, msg, stage, bvec,
                    shared, sem, sem2):
        cid = lax.axis_index("c")
        sid = lax.axis_index("s")
        zero16 = jnp.zeros((16,), F32)
        msk8 = lax.iota(jnp.int32, 16) < 8
        pltpu.sync_copy(bias_hbm, bvec)

        def qpass(q, h_hbm, o_hbm):
            def zrow(i, _):
                stage[i, :] = zero16
                return 0
            lax.fori_loop(0, HRPT, zrow, 0)
            pltpu.sync_copy(stage, shared.at[pl.ds(sid * RPT, HRPT)])
            pltpu.sync_copy(stage, shared.at[pl.ds(sid * RPT + HRPT, HRPT)])
            plsc.subcore_barrier()

            def chunk(g, _):
                base = sid * EPT + g * CH
                pltpu.sync_copy(src_hbm.at[pl.ds(base, CH)], sidx)
                pltpu.sync_copy(dst_hbm.at[pl.ds(base, CH)], didx)
                c1 = pltpu.async_copy(al_hbm.at[pl.ds(base, CH)], alv, sem)
                c4 = pltpu.async_copy(h_hbm.at[sidx], hv, sem2)
                c1.wait()
                c4.wait()

                def edge(i, _):
                    al = alv[i, :]
                    if H == 8:
                        a_lo = al[2 * q]
                        a_hi = al[2 * q + 1]
                        m = jnp.where(msk8, jnp.full((16,), a_lo, F32),
                                      jnp.full((16,), a_hi, F32))
                    else:
                        m = jnp.full((16,), al[0], F32)
                    msg[i, :] = hv[i, :] * m
                    return 0
                lax.fori_loop(0, CH, edge, 0)
                pltpu.sync_copy(msg, shared.at[didx], add=True)
                return 0
            lax.fori_loop(0, NCHM, chunk, 0)
            plsc.subcore_barrier()

            bq = bvec[pl.ds(16 * q, 16)]

            def dump(half, _):
                r0 = sid * RPT + half * HRPT
                pltpu.sync_copy(shared.at[pl.ds(r0, HRPT)], stage)

                def brow(i, _):
                    stage[i, :] = stage[i, :] + bq
                    return 0
                lax.fori_loop(0, HRPT, brow, 0)
                pltpu.sync_copy(stage, o_hbm.at[pl.ds(r0, HRPT)])
                return 0
            lax.fori_loop(0, 2, dump, 0)

        @pl.when(cid == 0)
        def _():
            qpass(0, h0_hbm, o0_hbm)
            qpass(1, h1_hbm, o1_hbm)

        @pl.when(cid == 1)
        def _():
            qpass(2, h2_hbm, o2_hbm)
            qpass(3, h3_hbm, o3_hbm)

    def make_msg(H):
        return functools.partial(
            pl.kernel, mesh=mesh, compiler_params=CP,
            out_type=(_f((NP, 16)),) * 4,
            scratch_types=[
                pltpu.VMEM((CH,), jnp.int32),
                pltpu.VMEM((CH,), jnp.int32),
                pltpu.VMEM((CH, 16), F32),
                pltpu.VMEM((CH, 16), F32),
                pltpu.VMEM((CH, 16), F32),
                pltpu.VMEM((HRPT, 16), F32),
                pltpu.VMEM((64,), F32),
                pltpu.VMEM_SHARED((NSP, 16), F32),
                pltpu.SemaphoreType.DMA,
                pltpu.SemaphoreType.DMA,
            ],
        )(functools.partial(sc_msg_body, H))

    sc_msg8 = make_msg(8)
    sc_msg1 = make_msg(1)

    return k1, k2, sc_den, sc_alpha, sc_msg8, sc_msg1


def _expand_heads(a):
    h = a.shape[0]
    if h == 1:
        m = jnp.tile(a.reshape(-1, 1), (1, 8))
    else:
        eye = jnp.eye(h, dtype=F32)
        m = (eye[:, None, :] * a[:, :, None]).reshape(-1, h)
    return jnp.concatenate([m, jnp.zeros_like(m)], axis=1)


def kernel(x, edge_index, W1, a_src1, a_dst1, b1, W2, a_src2, a_dst2, b2):
    k1, k2, sc_den, sc_alpha, sc_msg8, sc_msg1 = _build()
    N, E = N_NODES, N_EDGES

    src = edge_index[0].astype(jnp.int32)
    dst = edge_index[1].astype(jnp.int32)
    loop = jnp.arange(N, dtype=jnp.int32)
    npad = EP - E - N
    srcp = jnp.concatenate([src, loop, jnp.zeros((npad,), jnp.int32)])
    dstp = jnp.concatenate([dst, loop, jnp.full((npad,), N, jnp.int32)])
    xp = jnp.pad(x, ((0, NP - N), (0, 0)))

    h0, h1, h2, h3, as1, ad1 = k1(xp, W1, _expand_heads(a_src1),
                                  _expand_heads(a_dst1))
    d0, d1 = sc_den(srcp, dstp, as1, ad1)
    al1 = sc_alpha(srcp, dstp, as1, ad1, d0, d1)
    zeros64 = jnp.zeros((64,), F32)
    o0, o1, o2, o3 = sc_msg8(srcp, dstp, al1, h0, h1, h2, h3, zeros64)

    g0, g1, g2, g3, as2, ad2 = k2(o0, o1, o2, o3, b1.reshape(1, 64), W2,
                                  _expand_heads(a_src2), _expand_heads(a_dst2))
    d0b, d1b = sc_den(srcp, dstp, as2, ad2)
    al2 = sc_alpha(srcp, dstp, as2, ad2, d0b, d1b)
    p0, p1, p2, p3 = sc_msg1(srcp, dstp, al2, g0, g1, g2, g3, b2)

    return jnp.concatenate([p0[:N], p1[:N], p2[:N], p3[:N]], axis=1)


# speedup vs baseline: 1.4182x; 1.0552x over previous
import functools

import jax
import jax.numpy as jnp
from jax import lax
from jax.experimental import pallas as pl
from jax.experimental.pallas import tpu as pltpu
from jax.experimental.pallas import tpu_sc as plsc

N_NODES = 50000
N_EDGES = 800000
NP = 51200
EP = 851968
CH = 1024
CM = 512
NSP = 50016
RPT = NSP // 16
HRPT = RPT // 2
F32 = jnp.float32


def _f(shape):
    return jax.ShapeDtypeStruct(shape, F32)


@functools.lru_cache(maxsize=None)
def _build():
    info = plsc.get_sparse_core_info()
    NC, NS = info.num_cores, info.num_subcores
    NW = NC * NS
    mesh = plsc.VectorSubcoreMesh(core_axis_name="c", subcore_axis_name="s")
    CP = pltpu.CompilerParams(use_tc_tiling_on_sc=False)

    def k1_body(x_ref, w_ref, aps_ref, apd_ref,
                h0_ref, h1_ref, h2_ref, h3_ref, as_ref, ad_ref):
        h = jnp.dot(x_ref[...], w_ref[...], preferred_element_type=F32)
        h0_ref[...] = h[:, 0:16]
        h1_ref[...] = h[:, 16:32]
        h2_ref[...] = h[:, 32:48]
        h3_ref[...] = h[:, 48:64]
        as_ref[...] = jnp.dot(h, aps_ref[...], preferred_element_type=F32)
        ad_ref[...] = jnp.dot(h, apd_ref[...], preferred_element_type=F32)

    blk16 = pl.BlockSpec((512, 16), lambda i: (i, 0))
    k1 = pl.pallas_call(
        k1_body,
        grid=(NP // 512,),
        in_specs=[
            pl.BlockSpec((512, 300), lambda i: (i, 0)),
            pl.BlockSpec((300, 64), lambda i: (0, 0)),
            pl.BlockSpec((64, 16), lambda i: (0, 0)),
            pl.BlockSpec((64, 16), lambda i: (0, 0)),
        ],
        out_specs=[blk16] * 6,
        out_shape=[_f((NP, 16))] * 6,
    )

    def k2_body(o0_ref, o1_ref, o2_ref, o3_ref, b_ref, w_ref, aps_ref,
                apd_ref, h0_ref, h1_ref, h2_ref, h3_ref, as_ref, ad_ref):
        h = jnp.concatenate(
            [o0_ref[...], o1_ref[...], o2_ref[...], o3_ref[...]], axis=1)
        h = h + b_ref[...]
        z = jnp.where(h > 0, h, jnp.exp(h) - 1.0)
        h2 = jnp.dot(z, w_ref[...], preferred_element_type=F32)
        h0_ref[...] = h2[:, 0:16]
        h1_ref[...] = h2[:, 16:32]
        h2_ref[...] = h2[:, 32:48]
        h3_ref[...] = h2[:, 48:64]
        as_ref[...] = jnp.dot(h2, aps_ref[...], preferred_element_type=F32)
        ad_ref[...] = jnp.dot(h2, apd_ref[...], preferred_element_type=F32)

    k2 = pl.pallas_call(
        k2_body,
        grid=(NP // 512,),
        in_specs=[
            blk16, blk16, blk16, blk16,
            pl.BlockSpec((1, 64), lambda i: (0, 0)),
            pl.BlockSpec((64, 64), lambda i: (0, 0)),
            pl.BlockSpec((64, 16), lambda i: (0, 0)),
            pl.BlockSpec((64, 16), lambda i: (0, 0)),
        ],
        out_specs=[blk16] * 6,
        out_shape=[_f((NP, 16))] * 6,
    )

    def kc_body(d0_ref, d1_ref, rd_ref):
        rd_ref[...] = 1.0 / (d0_ref[...] + d1_ref[...] + 1e-16)

    kcomb = pl.pallas_call(
        kc_body,
        grid=(NP // 2048,),
        in_specs=[pl.BlockSpec((2048, 16), lambda i: (i, 0))] * 2,
        out_specs=pl.BlockSpec((2048, 16), lambda i: (i, 0)),
        out_shape=_f((NP, 16)),
    )

    EPW = EP // NW
    NCHD = EPW // CH

    @functools.partial(
        pl.kernel, mesh=mesh, compiler_params=CP,
        out_type=(_f((NP, 16)), _f((NP, 16))),
        scratch_types=[
            pltpu.VMEM((CH,), jnp.int32),
            pltpu.VMEM((CH,), jnp.int32),
            pltpu.VMEM((CH, 16), F32),
            pltpu.VMEM((CH, 16), F32),
            pltpu.VMEM((CH, 16), F32),
            pltpu.VMEM((HRPT, 16), F32),
            pltpu.VMEM_SHARED((NSP, 16), F32),
            pltpu.SemaphoreType.DMA,
        ],
    )
    def sc_den(src_hbm, dst_hbm, as_hbm, ad_hbm, d0_hbm, d1_hbm,
               sidx, didx, rs, rd, wv, stage, shared, sem):
        cid = lax.axis_index("c")
        sid = lax.axis_index("s")
        wid = sid * NC + cid
        zero16 = jnp.zeros((16,), F32)

        def zrow(i, _):
            stage[i, :] = zero16
            return 0
        lax.fori_loop(0, HRPT, zrow, 0)
        pltpu.sync_copy(stage, shared.at[pl.ds(sid * RPT, HRPT)])
        pltpu.sync_copy(stage, shared.at[pl.ds(sid * RPT + HRPT, HRPT)])
        plsc.subcore_barrier()

        def chunk(g, _):
            base = wid * EPW + g * CH
            pltpu.sync_copy(src_hbm.at[pl.ds(base, CH)], sidx)
            pltpu.sync_copy(dst_hbm.at[pl.ds(base, CH)], didx)
            c1 = pltpu.async_copy(as_hbm.at[sidx], rs, sem)
            c2 = pltpu.async_copy(ad_hbm.at[didx], rd, sem)
            c1.wait()
            c2.wait()

            def edge(i, _):
                e = rs[i, :] + rd[i, :]
                e = jnp.where(e < 0, e * jnp.float32(0.2), e)
                wv[i, :] = jnp.exp(e)
                return 0
            lax.fori_loop(0, CH, edge, 0)
            pltpu.sync_copy(wv, shared.at[didx], add=True)
            return 0
        lax.fori_loop(0, NCHD, chunk, 0)
        plsc.subcore_barrier()

        def dump(half, _):
            r0 = sid * RPT + half * HRPT
            pltpu.sync_copy(shared.at[pl.ds(r0, HRPT)], stage)

            @pl.when(cid == 0)
            def _():
                pltpu.sync_copy(stage, d0_hbm.at[pl.ds(r0, HRPT)])

            @pl.when(cid == 1)
            def _():
                pltpu.sync_copy(stage, d1_hbm.at[pl.ds(r0, HRPT)])
            return 0
        lax.fori_loop(0, 2, dump, 0)

    @functools.partial(
        pl.kernel, mesh=mesh, compiler_params=CP,
        out_type=_f((EP, 16)),
        scratch_types=[
            pltpu.VMEM((CH,), jnp.int32),
            pltpu.VMEM((CH,), jnp.int32),
            pltpu.VMEM((CH, 16), F32),
            pltpu.VMEM((CH, 16), F32),
            pltpu.VMEM((CH, 16), F32),
            pltpu.VMEM((CH, 16), F32),
            pltpu.VMEM((CH, 16), F32),
            pltpu.SemaphoreType.DMA,
        ],
    )
    def sc_alpha(src_hbm, dst_hbm, as_hbm, ad_hbm, d0_hbm, d1_hbm, al_hbm,
                 sidx, didx, rs, rd, rdn, dn1, alv, sem):
        cid = lax.axis_index("c")
        sid = lax.axis_index("s")
        wid = sid * NC + cid

        def chunk(g, _):
            base = wid * EPW + g * CH
            pltpu.sync_copy(src_hbm.at[pl.ds(base, CH)], sidx)
            pltpu.sync_copy(dst_hbm.at[pl.ds(base, CH)], didx)
            c1 = pltpu.async_copy(as_hbm.at[sidx], rs, sem)
            c2 = pltpu.async_copy(ad_hbm.at[didx], rd, sem)
            c3 = pltpu.async_copy(d0_hbm.at[didx], rdn, sem)
            c4 = pltpu.async_copy(d1_hbm.at[didx], dn1, sem)
            c1.wait()
            c2.wait()
            c3.wait()
            c4.wait()

            def edge(i, _):
                e = rs[i, :] + rd[i, :]
                e = jnp.where(e < 0, e * jnp.float32(0.2), e)
                den = rdn[i, :] + dn1[i, :] + jnp.float32(1e-16)
                alv[i, :] = jnp.exp(e) / den
                return 0
            lax.fori_loop(0, CH, edge, 0)
            pltpu.sync_copy(alv, al_hbm.at[pl.ds(base, CH)])
            return 0
        lax.fori_loop(0, NCHD, chunk, 0)

    EPT = EP // NS
    NCHM = EPT // CM

    def sc_msg_body(H, src_hbm, dst_hbm, al_hbm, h0_hbm, h1_hbm, h2_hbm,
                    h3_hbm, bias_hbm, o0_hbm, o1_hbm, o2_hbm, o3_hbm,
                    sidx0, sidx1, didx0, didx1, alv0, alv1, hv0, hv1,
                    msg, stage, bvec, shared, sem, sem2):
        cid = lax.axis_index("c")
        sid = lax.axis_index("s")
        zero16 = jnp.zeros((16,), F32)
        msk8 = lax.iota(jnp.int32, 16) < 8
        pltpu.sync_copy(bias_hbm, bvec)

        sidx = [sidx0, sidx1]
        didx = [didx0, didx1]
        alv = [alv0, alv1]
        hv = [hv0, hv1]
        sems = [sem, sem2]
        NPAIR = NCHM // 2

        def qpass(q, h_hbm, o_hbm):
            def zrow(i, _):
                stage[i, :] = zero16
                return 0
            lax.fori_loop(0, HRPT, zrow, 0)
            pltpu.sync_copy(stage, shared.at[pl.ds(sid * RPT, HRPT)])
            pltpu.sync_copy(stage, shared.at[pl.ds(sid * RPT + HRPT, HRPT)])
            plsc.subcore_barrier()

            def fire(b, cidx):
                base = sid * EPT + cidx * CM
                pltpu.sync_copy(src_hbm.at[pl.ds(base, CM)], sidx[b])
                pltpu.sync_copy(dst_hbm.at[pl.ds(base, CM)], didx[b])
                pltpu.async_copy(al_hbm.at[pl.ds(base, CM)], alv[b], sems[b])
                pltpu.async_copy(h_hbm.at[sidx[b]], hv[b], sems[b])

            def drain_compute(b):
                pltpu.make_async_copy(
                    al_hbm.at[pl.ds(0, CM)], alv[b], sems[b]).wait()
                pltpu.make_async_copy(
                    al_hbm.at[pl.ds(0, CM)], hv[b], sems[b]).wait()

                def edge(i, _):
                    al = alv[b][i, :]
                    if H == 8:
                        a_lo = al[2 * q]
                        a_hi = al[2 * q + 1]
                        m = jnp.where(msk8, jnp.full((16,), a_lo, F32),
                                      jnp.full((16,), a_hi, F32))
                    else:
                        m = jnp.full((16,), al[0], F32)
                    msg[i, :] = hv[b][i, :] * m
                    return 0
                lax.fori_loop(0, CM, edge, 0)
                pltpu.sync_copy(msg, shared.at[didx[b]], add=True)

            fire(0, 0)

            def pair(gp, _):
                fire(1, 2 * gp + 1)
                drain_compute(0)

                @pl.when(gp < NPAIR - 1)
                def _():
                    fire(0, 2 * gp + 2)
                drain_compute(1)
                return 0
            lax.fori_loop(0, NPAIR, pair, 0)
            plsc.subcore_barrier()

            bq = bvec[pl.ds(16 * q, 16)]

            def dump(half, _):
                r0 = sid * RPT + half * HRPT
                pltpu.sync_copy(shared.at[pl.ds(r0, HRPT)], stage)

                def brow(i, _):
                    stage[i, :] = stage[i, :] + bq
                    return 0
                lax.fori_loop(0, HRPT, brow, 0)
                pltpu.sync_copy(stage, o_hbm.at[pl.ds(r0, HRPT)])
                return 0
            lax.fori_loop(0, 2, dump, 0)

        @pl.when(cid == 0)
        def _():
            qpass(0, h0_hbm, o0_hbm)
            qpass(1, h1_hbm, o1_hbm)

        @pl.when(cid == 1)
        def _():
            qpass(2, h2_hbm, o2_hbm)
            qpass(3, h3_hbm, o3_hbm)

    def make_msg(H):
        return functools.partial(
            pl.kernel, mesh=mesh, compiler_params=CP,
            out_type=(_f((NP, 16)),) * 4,
            scratch_types=[
                pltpu.VMEM((CM,), jnp.int32),
                pltpu.VMEM((CM,), jnp.int32),
                pltpu.VMEM((CM,), jnp.int32),
                pltpu.VMEM((CM,), jnp.int32),
                pltpu.VMEM((CM, 16), F32),
                pltpu.VMEM((CM, 16), F32),
                pltpu.VMEM((CM, 16), F32),
                pltpu.VMEM((CM, 16), F32),
                pltpu.VMEM((CM, 16), F32),
                pltpu.VMEM((HRPT, 16), F32),
                pltpu.VMEM((64,), F32),
                pltpu.VMEM_SHARED((NSP, 16), F32),
                pltpu.SemaphoreType.DMA,
                pltpu.SemaphoreType.DMA,
            ],
        )(functools.partial(sc_msg_body, H))

    sc_msg8 = make_msg(8)
    sc_msg1 = make_msg(1)

    return k1, k2, sc_den, sc_alpha, sc_msg8, sc_msg1


def _expand_heads(a):
    h = a.shape[0]
    if h == 1:
        m = jnp.tile(a.reshape(-1, 1), (1, 8))
    else:
        eye = jnp.eye(h, dtype=F32)
        m = (eye[:, None, :] * a[:, :, None]).reshape(-1, h)
    return jnp.concatenate([m, jnp.zeros_like(m)], axis=1)


def kernel(x, edge_index, W1, a_src1, a_dst1, b1, W2, a_src2, a_dst2, b2):
    k1, k2, sc_den, sc_alpha, sc_msg8, sc_msg1 = _build()
    N, E = N_NODES, N_EDGES

    src = edge_index[0].astype(jnp.int32)
    dst = edge_index[1].astype(jnp.int32)
    loop = jnp.arange(N, dtype=jnp.int32)
    npad = EP - E - N
    srcp = jnp.concatenate([src, loop, jnp.zeros((npad,), jnp.int32)])
    dstp = jnp.concatenate([dst, loop, jnp.full((npad,), N, jnp.int32)])
    xp = jnp.pad(x, ((0, NP - N), (0, 0)))

    h0, h1, h2, h3, as1, ad1 = k1(xp, W1, _expand_heads(a_src1),
                                  _expand_heads(a_dst1))
    d0, d1 = sc_den(srcp, dstp, as1, ad1)
    al1 = sc_alpha(srcp, dstp, as1, ad1, d0, d1)
    zeros64 = jnp.zeros((64,), F32)
    o0, o1, o2, o3 = sc_msg8(srcp, dstp, al1, h0, h1, h2, h3, zeros64)

    g0, g1, g2, g3, as2, ad2 = k2(o0, o1, o2, o3, b1.reshape(1, 64), W2,
                                  _expand_heads(a_src2), _expand_heads(a_dst2))
    d0b, d1b = sc_den(srcp, dstp, as2, ad2)
    al2 = sc_alpha(srcp, dstp, as2, ad2, d0b, d1b)
    p0, p1, p2, p3 = sc_msg1(srcp, dstp, al2, g0, g1, g2, g3, b2)

    return jnp.concatenate([p0[:N], p1[:N], p2[:N], p3[:N]], axis=1)


# speedup vs baseline: 1.7468x; 1.2317x over previous
import functools

import jax
import jax.numpy as jnp
from jax import lax
from jax.experimental import pallas as pl
from jax.experimental.pallas import tpu as pltpu
from jax.experimental.pallas import tpu_sc as plsc

N_NODES = 50000
N_EDGES = 800000
NP = 51200
EP = 851968
CH = 1024
CM = 512
NSP = 50016
RPT = NSP // 16
HRPT = RPT // 2
F32 = jnp.float32


def _f(shape):
    return jax.ShapeDtypeStruct(shape, F32)


@functools.lru_cache(maxsize=None)
def _build():
    info = plsc.get_sparse_core_info()
    NC, NS = info.num_cores, info.num_subcores
    NW = NC * NS
    mesh = plsc.VectorSubcoreMesh(core_axis_name="c", subcore_axis_name="s")
    CP = pltpu.CompilerParams(use_tc_tiling_on_sc=False)

    def k1_body(x_ref, w_ref, aps_ref, apd_ref,
                h0_ref, h1_ref, h2_ref, h3_ref, as_ref, ad_ref):
        h = jnp.dot(x_ref[...], w_ref[...], preferred_element_type=F32)
        h0_ref[...] = h[:, 0:16]
        h1_ref[...] = h[:, 16:32]
        h2_ref[...] = h[:, 32:48]
        h3_ref[...] = h[:, 48:64]
        as_ref[...] = jnp.dot(h, aps_ref[...], preferred_element_type=F32)
        ad_ref[...] = jnp.dot(h, apd_ref[...], preferred_element_type=F32)

    blk16 = pl.BlockSpec((512, 16), lambda i: (i, 0))
    k1 = pl.pallas_call(
        k1_body,
        grid=(NP // 512,),
        in_specs=[
            pl.BlockSpec((512, 300), lambda i: (i, 0)),
            pl.BlockSpec((300, 64), lambda i: (0, 0)),
            pl.BlockSpec((64, 16), lambda i: (0, 0)),
            pl.BlockSpec((64, 16), lambda i: (0, 0)),
        ],
        out_specs=[blk16] * 6,
        out_shape=[_f((NP, 16))] * 6,
    )

    def k2_body(o0_ref, o1_ref, o2_ref, o3_ref, b_ref, w_ref, aps_ref,
                apd_ref, h0_ref, h1_ref, h2_ref, h3_ref, as_ref, ad_ref):
        h = jnp.concatenate(
            [o0_ref[...], o1_ref[...], o2_ref[...], o3_ref[...]], axis=1)
        h = h + b_ref[...]
        z = jnp.where(h > 0, h, jnp.exp(h) - 1.0)
        h2 = jnp.dot(z, w_ref[...], preferred_element_type=F32)
        h0_ref[...] = h2[:, 0:16]
        h1_ref[...] = h2[:, 16:32]
        h2_ref[...] = h2[:, 32:48]
        h3_ref[...] = h2[:, 48:64]
        as_ref[...] = jnp.dot(h2, aps_ref[...], preferred_element_type=F32)
        ad_ref[...] = jnp.dot(h2, apd_ref[...], preferred_element_type=F32)

    k2 = pl.pallas_call(
        k2_body,
        grid=(NP // 512,),
        in_specs=[
            blk16, blk16, blk16, blk16,
            pl.BlockSpec((1, 64), lambda i: (0, 0)),
            pl.BlockSpec((64, 64), lambda i: (0, 0)),
            pl.BlockSpec((64, 16), lambda i: (0, 0)),
            pl.BlockSpec((64, 16), lambda i: (0, 0)),
        ],
        out_specs=[blk16] * 6,
        out_shape=[_f((NP, 16))] * 6,
    )

    def kc_body(d0_ref, d1_ref, rd_ref):
        rd_ref[...] = 1.0 / (d0_ref[...] + d1_ref[...] + 1e-16)

    kcomb = pl.pallas_call(
        kc_body,
        grid=(NP // 2048,),
        in_specs=[pl.BlockSpec((2048, 16), lambda i: (i, 0))] * 2,
        out_specs=pl.BlockSpec((2048, 16), lambda i: (i, 0)),
        out_shape=_f((NP, 16)),
    )

    EPW = EP // NW
    NCHD = EPW // CH

    @functools.partial(
        pl.kernel, mesh=mesh, compiler_params=CP,
        out_type=(_f((NP, 16)), _f((NP, 16))),
        scratch_types=[
            pltpu.VMEM((CH,), jnp.int32),
            pltpu.VMEM((CH,), jnp.int32),
            pltpu.VMEM((CH, 16), F32),
            pltpu.VMEM((CH, 16), F32),
            pltpu.VMEM((CH, 16), F32),
            pltpu.VMEM((HRPT, 16), F32),
            pltpu.VMEM_SHARED((NSP, 16), F32),
            pltpu.SemaphoreType.DMA,
        ],
    )
    def sc_den(src_hbm, dst_hbm, as_hbm, ad_hbm, d0_hbm, d1_hbm,
               sidx, didx, rs, rd, wv, stage, shared, sem):
        cid = lax.axis_index("c")
        sid = lax.axis_index("s")
        wid = sid * NC + cid
        zero16 = jnp.zeros((16,), F32)

        def zrow(i, _):
            stage[i, :] = zero16
            return 0
        lax.fori_loop(0, HRPT, zrow, 0)
        pltpu.sync_copy(stage, shared.at[pl.ds(sid * RPT, HRPT)])
        pltpu.sync_copy(stage, shared.at[pl.ds(sid * RPT + HRPT, HRPT)])
        plsc.subcore_barrier()

        def chunk(g, _):
            base = wid * EPW + g * CH
            pltpu.sync_copy(src_hbm.at[pl.ds(base, CH)], sidx)
            pltpu.sync_copy(dst_hbm.at[pl.ds(base, CH)], didx)
            c1 = pltpu.async_copy(as_hbm.at[sidx], rs, sem)
            c2 = pltpu.async_copy(ad_hbm.at[didx], rd, sem)
            c1.wait()
            c2.wait()

            def edge(i, _):
                e = rs[i, :] + rd[i, :]
                e = jnp.where(e < 0, e * jnp.float32(0.2), e)
                wv[i, :] = jnp.exp(e)
                return 0
            lax.fori_loop(0, CH, edge, 0)
            pltpu.sync_copy(wv, shared.at[didx], add=True)
            return 0
        lax.fori_loop(0, NCHD, chunk, 0)
        plsc.subcore_barrier()

        def dump(half, _):
            r0 = sid * RPT + half * HRPT
            pltpu.sync_copy(shared.at[pl.ds(r0, HRPT)], stage)

            @pl.when(cid == 0)
            def _():
                pltpu.sync_copy(stage, d0_hbm.at[pl.ds(r0, HRPT)])

            @pl.when(cid == 1)
            def _():
                pltpu.sync_copy(stage, d1_hbm.at[pl.ds(r0, HRPT)])
            return 0
        lax.fori_loop(0, 2, dump, 0)

    @functools.partial(
        pl.kernel, mesh=mesh, compiler_params=CP,
        out_type=_f((EP, 16)),
        scratch_types=[
            pltpu.VMEM((CH,), jnp.int32),
            pltpu.VMEM((CH,), jnp.int32),
            pltpu.VMEM((CH, 16), F32),
            pltpu.VMEM((CH, 16), F32),
            pltpu.VMEM((CH, 16), F32),
            pltpu.VMEM((CH, 16), F32),
            pltpu.VMEM((CH, 16), F32),
            pltpu.SemaphoreType.DMA,
        ],
    )
    def sc_alpha(src_hbm, dst_hbm, as_hbm, ad_hbm, d0_hbm, d1_hbm, al_hbm,
                 sidx, didx, rs, rd, rdn, dn1, alv, sem):
        cid = lax.axis_index("c")
        sid = lax.axis_index("s")
        wid = sid * NC + cid

        def chunk(g, _):
            base = wid * EPW + g * CH
            pltpu.sync_copy(src_hbm.at[pl.ds(base, CH)], sidx)
            pltpu.sync_copy(dst_hbm.at[pl.ds(base, CH)], didx)
            c1 = pltpu.async_copy(as_hbm.at[sidx], rs, sem)
            c2 = pltpu.async_copy(ad_hbm.at[didx], rd, sem)
            c3 = pltpu.async_copy(d0_hbm.at[didx], rdn, sem)
            c4 = pltpu.async_copy(d1_hbm.at[didx], dn1, sem)
            c1.wait()
            c2.wait()
            c3.wait()
            c4.wait()

            def edge(i, _):
                e = rs[i, :] + rd[i, :]
                e = jnp.where(e < 0, e * jnp.float32(0.2), e)
                den = rdn[i, :] + dn1[i, :] + jnp.float32(1e-16)
                alv[i, :] = jnp.exp(e) / den
                return 0
            lax.fori_loop(0, CH, edge, 0)
            pltpu.sync_copy(alv, al_hbm.at[pl.ds(base, CH)])
            return 0
        lax.fori_loop(0, NCHD, chunk, 0)

    EPT = EP // NS
    NCHM = EPT // CM

    def sc_msg_body(H, src_hbm, dst_hbm, al_hbm, h0_hbm, h1_hbm, h2_hbm,
                    h3_hbm, bias_hbm, o0_hbm, o1_hbm, o2_hbm, o3_hbm,
                    sidx0, sidx1, didx0, didx1, alv0, alv1, hv0, hv1,
                    msg, stage, bvec, shared, sem, sem2):
        cid = lax.axis_index("c")
        sid = lax.axis_index("s")
        zero16 = jnp.zeros((16,), F32)
        msk8 = lax.iota(jnp.int32, 16) < 8
        pltpu.sync_copy(bias_hbm, bvec)

        sidx = [sidx0, sidx1]
        didx = [didx0, didx1]
        alv = [alv0, alv1]
        hv = [hv0, hv1]
        sems = [sem, sem2]
        NPAIR = NCHM // 2

        def qpass(q, h_hbm, o_hbm):
            def zrow(i, _):
                stage[i, :] = zero16
                return 0
            lax.fori_loop(0, HRPT, zrow, 0)
            pltpu.sync_copy(stage, shared.at[pl.ds(sid * RPT, HRPT)])
            pltpu.sync_copy(stage, shared.at[pl.ds(sid * RPT + HRPT, HRPT)])
            plsc.subcore_barrier()

            def fire(b, cidx):
                base = sid * EPT + cidx * CM
                pltpu.sync_copy(src_hbm.at[pl.ds(base, CM)], sidx[b])
                pltpu.sync_copy(dst_hbm.at[pl.ds(base, CM)], didx[b])
                pltpu.async_copy(al_hbm.at[pl.ds(base, CM)], alv[b], sems[b])
                pltpu.async_copy(h_hbm.at[sidx[b]], hv[b], sems[b])

            def drain_compute(b):
                pltpu.make_async_copy(
                    al_hbm.at[pl.ds(0, CM)], alv[b], sems[b]).wait()
                pltpu.make_async_copy(
                    al_hbm.at[pl.ds(0, CM)], hv[b], sems[b]).wait()

                @plsc.parallel_loop(0, CM, unroll=4)
                def _(i):
                    al = alv[b][i, :]
                    if H == 8:
                        a_lo = al[2 * q]
                        a_hi = al[2 * q + 1]
                        m = jnp.where(msk8, jnp.full((16,), a_lo, F32),
                                      jnp.full((16,), a_hi, F32))
                    else:
                        m = jnp.full((16,), al[0], F32)
                    msg[i, :] = hv[b][i, :] * m
                pltpu.sync_copy(msg, shared.at[didx[b]], add=True)

            fire(0, 0)

            def pair(gp, _):
                fire(1, 2 * gp + 1)
                drain_compute(0)

                @pl.when(gp < NPAIR - 1)
                def _():
                    fire(0, 2 * gp + 2)
                drain_compute(1)
                return 0
            lax.fori_loop(0, NPAIR, pair, 0)
            plsc.subcore_barrier()

            bq = bvec[pl.ds(16 * q, 16)]

            def dump(half, _):
                r0 = sid * RPT + half * HRPT
                pltpu.sync_copy(shared.at[pl.ds(r0, HRPT)], stage)

                def brow(i, _):
                    stage[i, :] = stage[i, :] + bq
                    return 0
                lax.fori_loop(0, HRPT, brow, 0)
                pltpu.sync_copy(stage, o_hbm.at[pl.ds(r0, HRPT)])
                return 0
            lax.fori_loop(0, 2, dump, 0)

        @pl.when(cid == 0)
        def _():
            qpass(0, h0_hbm, o0_hbm)
            qpass(1, h1_hbm, o1_hbm)

        @pl.when(cid == 1)
        def _():
            qpass(2, h2_hbm, o2_hbm)
            qpass(3, h3_hbm, o3_hbm)

    def make_msg(H):
        return functools.partial(
            pl.kernel, mesh=mesh, compiler_params=CP,
            out_type=(_f((NP, 16)),) * 4,
            scratch_types=[
                pltpu.VMEM((CM,), jnp.int32),
                pltpu.VMEM((CM,), jnp.int32),
                pltpu.VMEM((CM,), jnp.int32),
                pltpu.VMEM((CM,), jnp.int32),
                pltpu.VMEM((CM, 16), F32),
                pltpu.VMEM((CM, 16), F32),
                pltpu.VMEM((CM, 16), F32),
                pltpu.VMEM((CM, 16), F32),
                pltpu.VMEM((CM, 16), F32),
                pltpu.VMEM((HRPT, 16), F32),
                pltpu.VMEM((64,), F32),
                pltpu.VMEM_SHARED((NSP, 16), F32),
                pltpu.SemaphoreType.DMA,
                pltpu.SemaphoreType.DMA,
            ],
        )(functools.partial(sc_msg_body, H))

    sc_msg8 = make_msg(8)
    sc_msg1 = make_msg(1)

    return k1, k2, sc_den, sc_alpha, sc_msg8, sc_msg1


def _expand_heads(a):
    h = a.shape[0]
    if h == 1:
        m = jnp.tile(a.reshape(-1, 1), (1, 8))
    else:
        eye = jnp.eye(h, dtype=F32)
        m = (eye[:, None, :] * a[:, :, None]).reshape(-1, h)
    return jnp.concatenate([m, jnp.zeros_like(m)], axis=1)


def kernel(x, edge_index, W1, a_src1, a_dst1, b1, W2, a_src2, a_dst2, b2):
    k1, k2, sc_den, sc_alpha, sc_msg8, sc_msg1 = _build()
    N, E = N_NODES, N_EDGES

    src = edge_index[0].astype(jnp.int32)
    dst = edge_index[1].astype(jnp.int32)
    loop = jnp.arange(N, dtype=jnp.int32)
    npad = EP - E - N
    srcp = jnp.concatenate([src, loop, jnp.zeros((npad,), jnp.int32)])
    dstp = jnp.concatenate([dst, loop, jnp.full((npad,), N, jnp.int32)])
    xp = jnp.pad(x, ((0, NP - N), (0, 0)))

    h0, h1, h2, h3, as1, ad1 = k1(xp, W1, _expand_heads(a_src1),
                                  _expand_heads(a_dst1))
    d0, d1 = sc_den(srcp, dstp, as1, ad1)
    al1 = sc_alpha(srcp, dstp, as1, ad1, d0, d1)
    zeros64 = jnp.zeros((64,), F32)
    o0, o1, o2, o3 = sc_msg8(srcp, dstp, al1, h0, h1, h2, h3, zeros64)

    g0, g1, g2, g3, as2, ad2 = k2(o0, o1, o2, o3, b1.reshape(1, 64), W2,
                                  _expand_heads(a_src2), _expand_heads(a_dst2))
    d0b, d1b = sc_den(srcp, dstp, as2, ad2)
    al2 = sc_alpha(srcp, dstp, as2, ad2, d0b, d1b)
    p0, p1, p2, p3 = sc_msg1(srcp, dstp, al2, g0, g1, g2, g3, b2)

    return jnp.concatenate([p0[:N], p1[:N], p2[:N], p3[:N]], axis=1)


# speedup vs baseline: 1.8662x; 1.0684x over previous
import functools

import jax
import jax.numpy as jnp
from jax import lax
from jax.experimental import pallas as pl
from jax.experimental.pallas import tpu as pltpu
from jax.experimental.pallas import tpu_sc as plsc

N_NODES = 50000
N_EDGES = 800000
NP = 51200
EP = 851968
CH = 1024
CM = 512
NSP = 50016
RPT = NSP // 16
HRPT = RPT // 2
F32 = jnp.float32


def _f(shape):
    return jax.ShapeDtypeStruct(shape, F32)


@functools.lru_cache(maxsize=None)
def _build():
    info = plsc.get_sparse_core_info()
    NC, NS = info.num_cores, info.num_subcores
    NW = NC * NS
    mesh = plsc.VectorSubcoreMesh(core_axis_name="c", subcore_axis_name="s")
    CP = pltpu.CompilerParams(use_tc_tiling_on_sc=False)

    def k1_body(x_ref, w_ref, aps_ref, apd_ref,
                h0_ref, h1_ref, h2_ref, h3_ref, as_ref, ad_ref):
        h = jnp.dot(x_ref[...], w_ref[...], preferred_element_type=F32)
        h0_ref[...] = h[:, 0:16]
        h1_ref[...] = h[:, 16:32]
        h2_ref[...] = h[:, 32:48]
        h3_ref[...] = h[:, 48:64]
        as_ref[...] = jnp.dot(h, aps_ref[...], preferred_element_type=F32)
        ad_ref[...] = jnp.dot(h, apd_ref[...], preferred_element_type=F32)

    blk16 = pl.BlockSpec((512, 16), lambda i: (i, 0))
    k1 = pl.pallas_call(
        k1_body,
        grid=(NP // 512,),
        in_specs=[
            pl.BlockSpec((512, 300), lambda i: (i, 0)),
            pl.BlockSpec((300, 64), lambda i: (0, 0)),
            pl.BlockSpec((64, 16), lambda i: (0, 0)),
            pl.BlockSpec((64, 16), lambda i: (0, 0)),
        ],
        out_specs=[blk16] * 6,
        out_shape=[_f((NP, 16))] * 6,
    )

    def k2_body(o0_ref, o1_ref, o2_ref, o3_ref, b_ref, w_ref, aps_ref,
                apd_ref, h0_ref, h1_ref, h2_ref, h3_ref, as_ref, ad_ref):
        h = jnp.concatenate(
            [o0_ref[...], o1_ref[...], o2_ref[...], o3_ref[...]], axis=1)
        h = h + b_ref[...]
        z = jnp.where(h > 0, h, jnp.exp(h) - 1.0)
        h2 = jnp.dot(z, w_ref[...], preferred_element_type=F32)
        h0_ref[...] = h2[:, 0:16]
        h1_ref[...] = h2[:, 16:32]
        h2_ref[...] = h2[:, 32:48]
        h3_ref[...] = h2[:, 48:64]
        as_ref[...] = jnp.dot(h2, aps_ref[...], preferred_element_type=F32)
        ad_ref[...] = jnp.dot(h2, apd_ref[...], preferred_element_type=F32)

    k2 = pl.pallas_call(
        k2_body,
        grid=(NP // 512,),
        in_specs=[
            blk16, blk16, blk16, blk16,
            pl.BlockSpec((1, 64), lambda i: (0, 0)),
            pl.BlockSpec((64, 64), lambda i: (0, 0)),
            pl.BlockSpec((64, 16), lambda i: (0, 0)),
            pl.BlockSpec((64, 16), lambda i: (0, 0)),
        ],
        out_specs=[blk16] * 6,
        out_shape=[_f((NP, 16))] * 6,
    )

    def kc_body(d0_ref, d1_ref, rd_ref):
        rd_ref[...] = 1.0 / (d0_ref[...] + d1_ref[...] + 1e-16)

    kcomb = pl.pallas_call(
        kc_body,
        grid=(NP // 2048,),
        in_specs=[pl.BlockSpec((2048, 16), lambda i: (i, 0))] * 2,
        out_specs=pl.BlockSpec((2048, 16), lambda i: (i, 0)),
        out_shape=_f((NP, 16)),
    )

    EPW = EP // NW
    NCHD = EPW // CH

    @functools.partial(
        pl.kernel, mesh=mesh, compiler_params=CP,
        out_type=(_f((NP, 16)), _f((NP, 16))),
        scratch_types=[
            pltpu.VMEM((CH,), jnp.int32),
            pltpu.VMEM((CH,), jnp.int32),
            pltpu.VMEM((CH, 16), F32),
            pltpu.VMEM((CH, 16), F32),
            pltpu.VMEM((CH, 16), F32),
            pltpu.VMEM((HRPT, 16), F32),
            pltpu.VMEM_SHARED((NSP, 16), F32),
            pltpu.SemaphoreType.DMA,
        ],
    )
    def sc_den(src_hbm, dst_hbm, as_hbm, ad_hbm, d0_hbm, d1_hbm,
               sidx, didx, rs, rd, wv, stage, shared, sem):
        cid = lax.axis_index("c")
        sid = lax.axis_index("s")
        wid = sid * NC + cid
        zero16 = jnp.zeros((16,), F32)

        def zrow(i, _):
            stage[i, :] = zero16
            return 0
        lax.fori_loop(0, HRPT, zrow, 0)
        pltpu.sync_copy(stage, shared.at[pl.ds(sid * RPT, HRPT)])
        pltpu.sync_copy(stage, shared.at[pl.ds(sid * RPT + HRPT, HRPT)])
        plsc.subcore_barrier()

        def chunk(g, _):
            base = wid * EPW + g * CH
            pltpu.sync_copy(src_hbm.at[pl.ds(base, CH)], sidx)
            pltpu.sync_copy(dst_hbm.at[pl.ds(base, CH)], didx)
            c1 = pltpu.async_copy(as_hbm.at[sidx], rs, sem)
            c2 = pltpu.async_copy(ad_hbm.at[didx], rd, sem)
            c1.wait()
            c2.wait()

            @plsc.parallel_loop(0, CH, unroll=4)
            def _(i):
                e = rs[i, :] + rd[i, :]
                e = jnp.where(e < 0, e * jnp.float32(0.2), e)
                wv[i, :] = jnp.exp(e)
            pltpu.sync_copy(wv, shared.at[didx], add=True)
            return 0
        lax.fori_loop(0, NCHD, chunk, 0)
        plsc.subcore_barrier()

        def dump(half, _):
            r0 = sid * RPT + half * HRPT
            pltpu.sync_copy(shared.at[pl.ds(r0, HRPT)], stage)

            @pl.when(cid == 0)
            def _():
                pltpu.sync_copy(stage, d0_hbm.at[pl.ds(r0, HRPT)])

            @pl.when(cid == 1)
            def _():
                pltpu.sync_copy(stage, d1_hbm.at[pl.ds(r0, HRPT)])
            return 0
        lax.fori_loop(0, 2, dump, 0)

    @functools.partial(
        pl.kernel, mesh=mesh, compiler_params=CP,
        out_type=_f((EP, 16)),
        scratch_types=[
            pltpu.VMEM((CH,), jnp.int32),
            pltpu.VMEM((CH,), jnp.int32),
            pltpu.VMEM((CH, 16), F32),
            pltpu.VMEM((CH, 16), F32),
            pltpu.VMEM((CH, 16), F32),
            pltpu.VMEM((CH, 16), F32),
            pltpu.VMEM((CH, 16), F32),
            pltpu.SemaphoreType.DMA,
        ],
    )
    def sc_alpha(src_hbm, dst_hbm, as_hbm, ad_hbm, d0_hbm, d1_hbm, al_hbm,
                 sidx, didx, rs, rd, rdn, dn1, alv, sem):
        cid = lax.axis_index("c")
        sid = lax.axis_index("s")
        wid = sid * NC + cid

        def chunk(g, _):
            base = wid * EPW + g * CH
            pltpu.sync_copy(src_hbm.at[pl.ds(base, CH)], sidx)
            pltpu.sync_copy(dst_hbm.at[pl.ds(base, CH)], didx)
            c1 = pltpu.async_copy(as_hbm.at[sidx], rs, sem)
            c2 = pltpu.async_copy(ad_hbm.at[didx], rd, sem)
            c3 = pltpu.async_copy(d0_hbm.at[didx], rdn, sem)
            c4 = pltpu.async_copy(d1_hbm.at[didx], dn1, sem)
            c1.wait()
            c2.wait()
            c3.wait()
            c4.wait()

            @plsc.parallel_loop(0, CH, unroll=4)
            def _(i):
                e = rs[i, :] + rd[i, :]
                e = jnp.where(e < 0, e * jnp.float32(0.2), e)
                den = rdn[i, :] + dn1[i, :] + jnp.float32(1e-16)
                alv[i, :] = jnp.exp(e) / den
            pltpu.sync_copy(alv, al_hbm.at[pl.ds(base, CH)])
            return 0
        lax.fori_loop(0, NCHD, chunk, 0)

    EPT = EP // NS
    NCHM = EPT // CM

    def sc_msg_body(H, src_hbm, dst_hbm, al_hbm, h0_hbm, h1_hbm, h2_hbm,
                    h3_hbm, bias_hbm, o0_hbm, o1_hbm, o2_hbm, o3_hbm,
                    sidx0, sidx1, didx0, didx1, alv0, alv1, hv0, hv1,
                    msg, stage, bvec, shared, sem, sem2):
        cid = lax.axis_index("c")
        sid = lax.axis_index("s")
        zero16 = jnp.zeros((16,), F32)
        msk8 = lax.iota(jnp.int32, 16) < 8
        pltpu.sync_copy(bias_hbm, bvec)

        sidx = [sidx0, sidx1]
        didx = [didx0, didx1]
        alv = [alv0, alv1]
        hv = [hv0, hv1]
        sems = [sem, sem2]
        NPAIR = NCHM // 2

        def qpass(q, h_hbm, o_hbm):
            def zrow(i, _):
                stage[i, :] = zero16
                return 0
            lax.fori_loop(0, HRPT, zrow, 0)
            pltpu.sync_copy(stage, shared.at[pl.ds(sid * RPT, HRPT)])
            pltpu.sync_copy(stage, shared.at[pl.ds(sid * RPT + HRPT, HRPT)])
            plsc.subcore_barrier()

            def fire(b, cidx):
                base = sid * EPT + cidx * CM
                pltpu.sync_copy(src_hbm.at[pl.ds(base, CM)], sidx[b])
                pltpu.sync_copy(dst_hbm.at[pl.ds(base, CM)], didx[b])
                pltpu.async_copy(al_hbm.at[pl.ds(base, CM)], alv[b], sems[b])
                pltpu.async_copy(h_hbm.at[sidx[b]], hv[b], sems[b])

            def drain_compute(b):
                pltpu.make_async_copy(
                    al_hbm.at[pl.ds(0, CM)], alv[b], sems[b]).wait()
                pltpu.make_async_copy(
                    al_hbm.at[pl.ds(0, CM)], hv[b], sems[b]).wait()

                @plsc.parallel_loop(0, CM, unroll=4)
                def _(i):
                    al = alv[b][i, :]
                    if H == 8:
                        a_lo = al[2 * q]
                        a_hi = al[2 * q + 1]
                        m = jnp.where(msk8, jnp.full((16,), a_lo, F32),
                                      jnp.full((16,), a_hi, F32))
                    else:
                        m = jnp.full((16,), al[0], F32)
                    msg[i, :] = hv[b][i, :] * m
                pltpu.sync_copy(msg, shared.at[didx[b]], add=True)

            fire(0, 0)

            def pair(gp, _):
                fire(1, 2 * gp + 1)
                drain_compute(0)

                @pl.when(gp < NPAIR - 1)
                def _():
                    fire(0, 2 * gp + 2)
                drain_compute(1)
                return 0
            lax.fori_loop(0, NPAIR, pair, 0)
            plsc.subcore_barrier()

            bq = bvec[pl.ds(16 * q, 16)]

            def dump(half, _):
                r0 = sid * RPT + half * HRPT
                pltpu.sync_copy(shared.at[pl.ds(r0, HRPT)], stage)

                def brow(i, _):
                    stage[i, :] = stage[i, :] + bq
                    return 0
                lax.fori_loop(0, HRPT, brow, 0)
                pltpu.sync_copy(stage, o_hbm.at[pl.ds(r0, HRPT)])
                return 0
            lax.fori_loop(0, 2, dump, 0)

        @pl.when(cid == 0)
        def _():
            qpass(0, h0_hbm, o0_hbm)
            qpass(1, h1_hbm, o1_hbm)

        @pl.when(cid == 1)
        def _():
            qpass(2, h2_hbm, o2_hbm)
            qpass(3, h3_hbm, o3_hbm)

    def make_msg(H):
        return functools.partial(
            pl.kernel, mesh=mesh, compiler_params=CP,
            out_type=(_f((NP, 16)),) * 4,
            scratch_types=[
                pltpu.VMEM((CM,), jnp.int32),
                pltpu.VMEM((CM,), jnp.int32),
                pltpu.VMEM((CM,), jnp.int32),
                pltpu.VMEM((CM,), jnp.int32),
                pltpu.VMEM((CM, 16), F32),
                pltpu.VMEM((CM, 16), F32),
                pltpu.VMEM((CM, 16), F32),
                pltpu.VMEM((CM, 16), F32),
                pltpu.VMEM((CM, 16), F32),
                pltpu.VMEM((HRPT, 16), F32),
                pltpu.VMEM((64,), F32),
                pltpu.VMEM_SHARED((NSP, 16), F32),
                pltpu.SemaphoreType.DMA,
                pltpu.SemaphoreType.DMA,
            ],
        )(functools.partial(sc_msg_body, H))

    sc_msg8 = make_msg(8)
    sc_msg1 = make_msg(1)

    return k1, k2, sc_den, sc_alpha, sc_msg8, sc_msg1


def _expand_heads(a):
    h = a.shape[0]
    if h == 1:
        m = jnp.tile(a.reshape(-1, 1), (1, 8))
    else:
        eye = jnp.eye(h, dtype=F32)
        m = (eye[:, None, :] * a[:, :, None]).reshape(-1, h)
    return jnp.concatenate([m, jnp.zeros_like(m)], axis=1)


def kernel(x, edge_index, W1, a_src1, a_dst1, b1, W2, a_src2, a_dst2, b2):
    k1, k2, sc_den, sc_alpha, sc_msg8, sc_msg1 = _build()
    N, E = N_NODES, N_EDGES

    src = edge_index[0].astype(jnp.int32)
    dst = edge_index[1].astype(jnp.int32)
    loop = jnp.arange(N, dtype=jnp.int32)
    npad = EP - E - N
    srcp = jnp.concatenate([src, loop, jnp.zeros((npad,), jnp.int32)])
    dstp = jnp.concatenate([dst, loop, jnp.full((npad,), N, jnp.int32)])
    xp = jnp.pad(x, ((0, NP - N), (0, 0)))

    h0, h1, h2, h3, as1, ad1 = k1(xp, W1, _expand_heads(a_src1),
                                  _expand_heads(a_dst1))
    d0, d1 = sc_den(srcp, dstp, as1, ad1)
    al1 = sc_alpha(srcp, dstp, as1, ad1, d0, d1)
    zeros64 = jnp.zeros((64,), F32)
    o0, o1, o2, o3 = sc_msg8(srcp, dstp, al1, h0, h1, h2, h3, zeros64)

    g0, g1, g2, g3, as2, ad2 = k2(o0, o1, o2, o3, b1.reshape(1, 64), W2,
                                  _expand_heads(a_src2), _expand_heads(a_dst2))
    d0b, d1b = sc_den(srcp, dstp, as2, ad2)
    al2 = sc_alpha(srcp, dstp, as2, ad2, d0b, d1b)
    p0, p1, p2, p3 = sc_msg1(srcp, dstp, al2, g0, g1, g2, g3, b2)

    return jnp.concatenate([p0[:N], p1[:N], p2[:N], p3[:N]], axis=1)
